# baseline (speedup 1.0000x reference)
import functools

import jax
import jax.numpy as jnp
from jax import lax
from jax.experimental import pallas as pl
from jax.experimental.pallas import tpu as pltpu
from jax.experimental.pallas import tpu_sc as plsc

N = 10000
E = 320000
D = 128
H = 32

NNP = 10016
TRASH = 10000
NW = 32
EPW = E // NW
CW = 128
CH = 80
EPW_PAD = CH * CW
RPS = NNP // 16

BR = 2000
NB = N // BR


YRS = N // 16


def _sc_agg_body(y_hbm, srcs_hbm, dsts_hbm, zeros_hbm, out_hbm,
                 src_v, dst_v, rows_v, y_sh, acc_sh):
    cid = lax.axis_index("c")
    sid = lax.axis_index("s")
    wid = sid * 2 + cid
    r0 = sid * RPS
    pltpu.sync_copy(zeros_hbm.at[pl.ds(r0, RPS)], acc_sh.at[pl.ds(r0, RPS)])
    pltpu.sync_copy(y_hbm.at[pl.ds(sid * YRS, YRS)],
                    y_sh.at[pl.ds(sid * YRS, YRS)])
    pltpu.sync_copy(srcs_hbm.at[wid], src_v)
    pltpu.sync_copy(dsts_hbm.at[wid], dst_v)
    plsc.subcore_barrier()

    def body(j, carry):
        pltpu.sync_copy(y_sh.at[src_v.at[j]], rows_v)
        pltpu.sync_copy(rows_v, acc_sh.at[dst_v.at[j]], add=True)
        return carry

    lax.fori_loop(0, CH, body, 0)
    plsc.subcore_barrier()
    pltpu.sync_copy(acc_sh.at[pl.ds(r0, RPS)],
                    out_hbm.at[cid, pl.ds(r0, RPS)])


_sc_agg = functools.partial(
    pl.kernel,
    mesh=plsc.VectorSubcoreMesh(core_axis_name="c", subcore_axis_name="s"),
    out_type=jax.ShapeDtypeStruct((2, NNP, H), jnp.float32),
    scratch_types=[
        pltpu.VMEM((CH, CW), jnp.int32),
        pltpu.VMEM((CH, CW), jnp.int32),
        pltpu.VMEM((CW, H), jnp.float32),
        pltpu.VMEM_SHARED((NNP, H), jnp.float32),
        pltpu.VMEM_SHARED((NNP, H), jnp.float32),
    ],
    compiler_params=pltpu.CompilerParams(use_tc_tiling_on_sc=False),
)(_sc_agg_body)


HC = 8


def _sc_cnt_body(ones_hbm, srcs_hbm, dsts_hbm, zeros_hbm, out_hbm,
                 src_v, dst_v, ones_v, accd_sh, accs_sh, sem):
    cid = lax.axis_index("c")
    sid = lax.axis_index("s")
    wid = sid * 2 + cid
    r0 = sid * RPS
    pltpu.sync_copy(zeros_hbm.at[pl.ds(r0, RPS)], accd_sh.at[pl.ds(r0, RPS)])
    pltpu.sync_copy(zeros_hbm.at[pl.ds(r0, RPS)], accs_sh.at[pl.ds(r0, RPS)])
    pltpu.sync_copy(srcs_hbm.at[wid], src_v)
    pltpu.sync_copy(dsts_hbm.at[wid], dst_v)
    pltpu.sync_copy(ones_hbm, ones_v)
    plsc.subcore_barrier()

    def body(j, carry):
        d1 = pltpu.async_copy(ones_v, accd_sh.at[dst_v.at[j]], sem, add=True)
        d2 = pltpu.async_copy(ones_v, accs_sh.at[src_v.at[j]], sem, add=True)
        d1.wait()
        d2.wait()
        return carry

    lax.fori_loop(0, CH, body, 0)
    plsc.subcore_barrier()
    pltpu.sync_copy(accd_sh.at[pl.ds(r0, RPS)],
                    out_hbm.at[cid, 0, pl.ds(r0, RPS)])
    pltpu.sync_copy(accs_sh.at[pl.ds(r0, RPS)],
                    out_hbm.at[cid, 1, pl.ds(r0, RPS)])


_sc_cnt = functools.partial(
    pl.kernel,
    mesh=plsc.VectorSubcoreMesh(core_axis_name="c", subcore_axis_name="s"),
    out_type=jax.ShapeDtypeStruct((2, 2, NNP, HC), jnp.float32),
    scratch_types=[
        pltpu.VMEM((CH, CW), jnp.int32),
        pltpu.VMEM((CH, CW), jnp.int32),
        pltpu.VMEM((CW, HC), jnp.float32),
        pltpu.VMEM_SHARED((NNP, HC), jnp.float32),
        pltpu.VMEM_SHARED((NNP, HC), jnp.float32),
        pltpu.SemaphoreType.DMA,
    ],
    compiler_params=pltpu.CompilerParams(use_tc_tiling_on_sc=False),
)(_sc_cnt_body)


_PREC = lax.Precision.HIGHEST
_FULL2 = lambda shape: pl.BlockSpec(shape, lambda p, j: tuple(0 for _ in shape))
_BLK2 = lambda w: pl.BlockSpec((BR, w), lambda p, j: (j, 0))


def _stats_accum(st_ref, h, cs, is_first):
    @pl.when(is_first)
    def _():
        st_ref[...] = jnp.zeros((D, 8), jnp.float32)
    s1 = lax.dot_general(h, cs, (((0,), (0,)), ((), ())), precision=_PREC)
    s2 = lax.dot_general(h * h, cs, (((0,), (0,)), ((), ())), precision=_PREC)
    st_ref[:, 0:1] += s1
    st_ref[:, 1:2] += s2


def _msg_math(h, st, g, b, Wsrc, bsrc):
    mu = st[:, 0:1] * (1.0 / E)
    var = st[:, 1:2] * (1.0 / E) - mu * mu
    scale = g * lax.rsqrt(var + 1e-5)
    shift = b - mu * scale
    Wp = scale * Wsrc
    bp = lax.dot_general(shift, Wsrc, (((0,), (0,)), ((), ())),
                         precision=_PREC) + bsrc
    z = jnp.dot(h, Wp, precision=_PREC) + bp
    return z * 0.5 * (1.0 + lax.erf(z * 0.7071067811865476))


def _tc_pre_body(x_ref, cntP_ref, g_ref, b_ref, Wsrc_ref, bsrc_ref,
                 y_ref, cd_ref, cs_ref, st_ref):
    p = pl.program_id(0)
    j = pl.program_id(1)
    cd = cntP_ref[0, 0, :, 0:1] + cntP_ref[1, 0, :, 0:1]
    cs = cntP_ref[0, 1, :, 0:1] + cntP_ref[1, 1, :, 0:1]
    cd_ref[...] = cd
    cs_ref[...] = cs

    @pl.when(p == 0)
    def _():
        _stats_accum(st_ref, x_ref[...], cs, j == 0)
        y_ref[...] = jnp.zeros((BR, H), jnp.float32)

    @pl.when(p == 1)
    def _():
        y_ref[...] = _msg_math(x_ref[...], st_ref[...], g_ref[...],
                               b_ref[...], Wsrc_ref[...], bsrc_ref[...])


def _tc_pre(x, cntP, g, b, Wsrc, bsrc):
    return pl.pallas_call(
        _tc_pre_body,
        grid=(2, NB),
        in_specs=[_BLK2(D),
                  pl.BlockSpec((2, 2, BR, HC), lambda p, j: (0, 0, j, 0)),
                  _FULL2((D, 1)), _FULL2((D, 1)),
                  _FULL2((D, H)), _FULL2((1, H))],
        out_specs=[_BLK2(H), _BLK2(1), _BLK2(1)],
        out_shape=[jax.ShapeDtypeStruct((N, H), jnp.float32),
                   jax.ShapeDtypeStruct((N, 1), jnp.float32),
                   jax.ShapeDtypeStruct((N, 1), jnp.float32)],
        scratch_shapes=[pltpu.VMEM((D, 8), jnp.float32)],
    )(x, cntP, g, b, Wsrc, bsrc)


def _apply_math(h_in, aggP, cd, Wfca, Wfcb, bfc, Wdst, bdst, apply_relu):
    agg = aggP[0] + aggP[1]
    neigh = agg / jnp.maximum(cd, 1.0)
    rst = (jnp.dot(h_in, Wfca, precision=_PREC)
           + jnp.dot(neigh, Wfcb, precision=_PREC) + bfc)
    if apply_relu:
        rst = jnp.maximum(rst, 0.0)
    return jnp.dot(h_in, Wdst, precision=_PREC) + bdst + rst


def _tc_mid_body(apply_relu,
                 h_ref, aggP_ref, cd_ref, cs_ref,
                 Wfca_ref, Wfcb_ref, bfc_ref, Wdst_ref, bdst_ref,
                 g_ref, b_ref, Wsrc_ref, bsrc_ref,
                 ho_ref, y_ref, st_ref, hs_ref):
    p = pl.program_id(0)
    j = pl.program_id(1)

    @pl.when(p == 0)
    def _():
        h = _apply_math(h_ref[...], aggP_ref[...], cd_ref[...],
                        Wfca_ref[...], Wfcb_ref[...], bfc_ref[...],
                        Wdst_ref[...], bdst_ref[...], apply_relu)
        ho_ref[...] = h
        hs_ref[pl.ds(j * BR, BR), :] = h
        _stats_accum(st_ref, h, cs_ref[...], j == 0)
        y_ref[...] = jnp.zeros((BR, H), jnp.float32)

    @pl.when(p == 1)
    def _():
        h = hs_ref[pl.ds(j * BR, BR), :]
        ho_ref[...] = h
        y_ref[...] = _msg_math(h, st_ref[...], g_ref[...], b_ref[...],
                               Wsrc_ref[...], bsrc_ref[...])


def _tc_mid(apply_relu, h, aggP, cd, cs, Wfca, Wfcb, bfc, Wdst, bdst,
            g, b, Wsrc, bsrc):
    return pl.pallas_call(
        functools.partial(_tc_mid_body, apply_relu),
        grid=(2, NB),
        in_specs=[_BLK2(D),
                  pl.BlockSpec((2, BR, H), lambda p, j: (0, j, 0)),
                  _BLK2(1), _BLK2(1),
                  _FULL2((D, D)), _FULL2((H, D)), _FULL2((1, D)),
                  _FULL2((D, D)), _FULL2((1, D)),
                  _FULL2((D, 1)), _FULL2((D, 1)),
                  _FULL2((D, H)), _FULL2((1, H))],
        out_specs=[_BLK2(D), _BLK2(H)],
        out_shape=[jax.ShapeDtypeStruct((N, D), jnp.float32),
                   jax.ShapeDtypeStruct((N, H), jnp.float32)],
        scratch_shapes=[pltpu.VMEM((D, 8), jnp.float32),
                        pltpu.VMEM((N, D), jnp.float32)],
    )(h, aggP, cd, cs, Wfca, Wfcb, bfc, Wdst, bdst, g, b, Wsrc, bsrc)


def _tc_fin_body(h_ref, aggP_ref, cd_ref,
                 Wfca_ref, Wfcb_ref, bfc_ref, Wdst_ref, bdst_ref, out_ref):
    out_ref[...] = _apply_math(h_ref[...], aggP_ref[...], cd_ref[...],
                               Wfca_ref[...], Wfcb_ref[...], bfc_ref[...],
                               Wdst_ref[...], bdst_ref[...], False)


def _tc_fin(h, aggP, cd, Wfca, Wfcb, bfc, Wdst, bdst):
    blk = lambda w: pl.BlockSpec((BR, w), lambda j: (j, 0))
    full = lambda shape: pl.BlockSpec(shape, lambda j: tuple(0 for _ in shape))
    return pl.pallas_call(
        _tc_fin_body,
        grid=(NB,),
        in_specs=[blk(D),
                  pl.BlockSpec((2, BR, H), lambda j: (0, j, 0)),
                  blk(1),
                  full((D, 1)), full((H, 1)), full((1, 1)),
                  full((D, 1)), full((1, 1))],
        out_specs=blk(1),
        out_shape=jax.ShapeDtypeStruct((N, 1), jnp.float32),
    )(h, aggP, cd, Wfca, Wfcb, bfc, Wdst, bdst)


@jax.jit
def kernel(x, edge_index,
           bn_g1, bn_b1, Wsrc1, bsrc1, Wfc1, bfc1, Wdst1, bdst1,
           bn_g2, bn_b2, Wsrc2, bsrc2, Wfc2, bfc2, Wdst2, bdst2,
           bn_g3, bn_b3, Wsrc3, bsrc3, Wfc3, bfc3, Wdst3, bdst3):
    src = edge_index[0]
    dst = edge_index[1]
    pspread = jnp.arange(EPW_PAD - EPW, dtype=jnp.int32)[None, :] % 16
    spad = pspread
    tpad = TRASH + pspread
    spad = jnp.broadcast_to(spad, (NW, EPW_PAD - EPW))
    tpad = jnp.broadcast_to(tpad, (NW, EPW_PAD - EPW))
    srcs_g = jnp.concatenate([src.reshape(NW, EPW), spad], axis=1)
    srcs_g = srcs_g.reshape(NW, CH, CW)
    srcs_c = jnp.concatenate([src.reshape(NW, EPW), tpad], axis=1)
    srcs_c = srcs_c.reshape(NW, CH, CW)
    dsts_r = jnp.concatenate([dst.reshape(NW, EPW), tpad], axis=1)
    dsts_r = dsts_r.reshape(NW, CH, CW)

    zeros = jnp.zeros((NNP, H), jnp.float32)
    zeros_n = jnp.zeros((NNP, HC), jnp.float32)
    ones_rows = jnp.ones((CW, HC), jnp.float32)

    cntP = _sc_cnt(ones_rows, srcs_c, dsts_r, zeros_n)

    g1 = bn_g1.reshape(D, 1); b1 = bn_b1.reshape(D, 1)
    g2 = bn_g2.reshape(D, 1); b2 = bn_b2.reshape(D, 1)
    g3 = bn_g3.reshape(D, 1); b3 = bn_b3.reshape(D, 1)

    y1, cd, cs = _tc_pre(x, cntP, g1, b1, Wsrc1, bsrc1.reshape(1, H))
    aggP1 = _sc_agg(y1, srcs_g, dsts_r, zeros)
    h1, y2 = _tc_mid(True, x, aggP1, cd, cs,
                     Wfc1[0:D], Wfc1[D:], bfc1.reshape(1, D),
                     Wdst1, bdst1.reshape(1, D),
                     g2, b2, Wsrc2, bsrc2.reshape(1, H))
    aggP2 = _sc_agg(y2, srcs_g, dsts_r, zeros)
    h2, y3 = _tc_mid(False, h1, aggP2, cd, cs,
                     Wfc2[0:D], Wfc2[D:], bfc2.reshape(1, D),
                     Wdst2, bdst2.reshape(1, D),
                     g3, b3, Wsrc3, bsrc3.reshape(1, H))
    aggP3 = _sc_agg(y3, srcs_g, dsts_r, zeros)
    out = _tc_fin(h2, aggP3, cd,
                  Wfc3[0:D], Wfc3[D:], bfc3.reshape(1, 1),
                  Wdst3, bdst3.reshape(1, 1))
    return out


# speedup vs baseline: 1.1642x; 1.1642x over previous
import functools

import jax
import jax.numpy as jnp
from jax import lax
from jax.experimental import pallas as pl
from jax.experimental.pallas import tpu as pltpu
from jax.experimental.pallas import tpu_sc as plsc

N = 10000
E = 320000
D = 128
H = 32

NNP = 10016
TRASH = 10000
NW = 32
EPW = E // NW
CW = 128
CH = 80
EPW_PAD = CH * CW
RPS = NNP // 16

BR = 2000
NB = N // BR


YRS = N // 16


def _sc_agg_body(y_hbm, srcs_hbm, dsts_hbm, zeros_hbm, out_hbm,
                 src_v, dst_v, rows_v, y_sh, acc_sh):
    cid = lax.axis_index("c")
    sid = lax.axis_index("s")
    wid = sid * 2 + cid
    r0 = sid * RPS
    pltpu.sync_copy(zeros_hbm.at[pl.ds(r0, RPS)], acc_sh.at[pl.ds(r0, RPS)])
    pltpu.sync_copy(y_hbm.at[pl.ds(sid * YRS, YRS)],
                    y_sh.at[pl.ds(sid * YRS, YRS)])
    pltpu.sync_copy(srcs_hbm.at[wid], src_v)
    pltpu.sync_copy(dsts_hbm.at[wid], dst_v)
    plsc.subcore_barrier()

    def body(j, carry):
        pltpu.sync_copy(y_sh.at[src_v.at[j]], rows_v)
        pltpu.sync_copy(rows_v, acc_sh.at[dst_v.at[j]], add=True)
        return carry

    lax.fori_loop(0, CH, body, 0)
    plsc.subcore_barrier()
    pltpu.sync_copy(acc_sh.at[pl.ds(r0, RPS)],
                    out_hbm.at[cid, pl.ds(r0, RPS)])


_sc_agg = functools.partial(
    pl.kernel,
    mesh=plsc.VectorSubcoreMesh(core_axis_name="c", subcore_axis_name="s"),
    out_type=jax.ShapeDtypeStruct((2, NNP, H), jnp.float32),
    scratch_types=[
        pltpu.VMEM((CH, CW), jnp.int32),
        pltpu.VMEM((CH, CW), jnp.int32),
        pltpu.VMEM((CW, H), jnp.float32),
        pltpu.VMEM_SHARED((NNP, H), jnp.float32),
        pltpu.VMEM_SHARED((NNP, H), jnp.float32),
    ],
    compiler_params=pltpu.CompilerParams(use_tc_tiling_on_sc=False),
)(_sc_agg_body)


HC = 8


def _sc_cnt_body(ones_hbm, srcs_hbm, dsts_hbm, zeros_hbm, out_hbm,
                 src_v, dst_v, ones_v, accd_sh, accs_sh, sem):
    cid = lax.axis_index("c")
    sid = lax.axis_index("s")
    wid = sid * 2 + cid
    r0 = sid * RPS
    pltpu.sync_copy(zeros_hbm.at[pl.ds(r0, RPS)], accd_sh.at[pl.ds(r0, RPS)])
    pltpu.sync_copy(zeros_hbm.at[pl.ds(r0, RPS)], accs_sh.at[pl.ds(r0, RPS)])
    pltpu.sync_copy(srcs_hbm.at[wid], src_v)
    pltpu.sync_copy(dsts_hbm.at[wid], dst_v)
    pltpu.sync_copy(ones_hbm, ones_v)
    plsc.subcore_barrier()

    def body(j, carry):
        d1 = pltpu.async_copy(ones_v, accd_sh.at[dst_v.at[j]], sem, add=True)
        d2 = pltpu.async_copy(ones_v, accs_sh.at[src_v.at[j]], sem, add=True)
        d1.wait()
        d2.wait()
        return carry

    lax.fori_loop(0, CH, body, 0)
    plsc.subcore_barrier()
    pltpu.sync_copy(accd_sh.at[pl.ds(r0, RPS)],
                    out_hbm.at[cid, 0, pl.ds(r0, RPS)])
    pltpu.sync_copy(accs_sh.at[pl.ds(r0, RPS)],
                    out_hbm.at[cid, 1, pl.ds(r0, RPS)])


_sc_cnt = functools.partial(
    pl.kernel,
    mesh=plsc.VectorSubcoreMesh(core_axis_name="c", subcore_axis_name="s"),
    out_type=jax.ShapeDtypeStruct((2, 2, NNP, HC), jnp.float32),
    scratch_types=[
        pltpu.VMEM((CH, CW), jnp.int32),
        pltpu.VMEM((CH, CW), jnp.int32),
        pltpu.VMEM((CW, HC), jnp.float32),
        pltpu.VMEM_SHARED((NNP, HC), jnp.float32),
        pltpu.VMEM_SHARED((NNP, HC), jnp.float32),
        pltpu.SemaphoreType.DMA,
    ],
    compiler_params=pltpu.CompilerParams(use_tc_tiling_on_sc=False),
)(_sc_cnt_body)


_PREC = lax.Precision.DEFAULT
_FULL2 = lambda shape: pl.BlockSpec(shape, lambda p, j: tuple(0 for _ in shape))
_BLK2 = lambda w: pl.BlockSpec((BR, w), lambda p, j: (j, 0))


def _stats_accum(st_ref, h, cs, is_first):
    @pl.when(is_first)
    def _():
        st_ref[...] = jnp.zeros((D, 8), jnp.float32)
    s1 = lax.dot_general(h, cs, (((0,), (0,)), ((), ())), precision=_PREC)
    s2 = lax.dot_general(h * h, cs, (((0,), (0,)), ((), ())), precision=_PREC)
    st_ref[:, 0:1] += s1
    st_ref[:, 1:2] += s2


def _msg_math(h, st, g, b, Wsrc, bsrc):
    mu = st[:, 0:1] * (1.0 / E)
    var = st[:, 1:2] * (1.0 / E) - mu * mu
    scale = g * lax.rsqrt(var + 1e-5)
    shift = b - mu * scale
    Wp = scale * Wsrc
    bp = lax.dot_general(shift, Wsrc, (((0,), (0,)), ((), ())),
                         precision=_PREC) + bsrc
    z = jnp.dot(h, Wp, precision=_PREC) + bp
    return z * 0.5 * (1.0 + lax.erf(z * 0.7071067811865476))


def _tc_pre_body(x_ref, cntP_ref, g_ref, b_ref, Wsrc_ref, bsrc_ref,
                 y_ref, cd_ref, cs_ref, st_ref):
    p = pl.program_id(0)
    j = pl.program_id(1)
    cd = cntP_ref[0, 0, :, 0:1] + cntP_ref[1, 0, :, 0:1]
    cs = cntP_ref[0, 1, :, 0:1] + cntP_ref[1, 1, :, 0:1]
    cd_ref[...] = cd
    cs_ref[...] = cs

    @pl.when(p == 0)
    def _():
        _stats_accum(st_ref, x_ref[...], cs, j == 0)
        y_ref[...] = jnp.zeros((BR, H), jnp.float32)

    @pl.when(p == 1)
    def _():
        y_ref[...] = _msg_math(x_ref[...], st_ref[...], g_ref[...],
                               b_ref[...], Wsrc_ref[...], bsrc_ref[...])


def _tc_pre(x, cntP, g, b, Wsrc, bsrc):
    return pl.pallas_call(
        _tc_pre_body,
        grid=(2, NB),
        in_specs=[_BLK2(D),
                  pl.BlockSpec((2, 2, BR, HC), lambda p, j: (0, 0, j, 0)),
                  _FULL2((D, 1)), _FULL2((D, 1)),
                  _FULL2((D, H)), _FULL2((1, H))],
        out_specs=[_BLK2(H), _BLK2(1), _BLK2(1)],
        out_shape=[jax.ShapeDtypeStruct((N, H), jnp.float32),
                   jax.ShapeDtypeStruct((N, 1), jnp.float32),
                   jax.ShapeDtypeStruct((N, 1), jnp.float32)],
        scratch_shapes=[pltpu.VMEM((D, 8), jnp.float32)],
    )(x, cntP, g, b, Wsrc, bsrc)


def _apply_math(h_in, aggP, cd, Wfca, Wfcb, bfc, Wdst, bdst, apply_relu):
    agg = aggP[0] + aggP[1]
    neigh = agg / jnp.maximum(cd, 1.0)
    rst = (jnp.dot(h_in, Wfca, precision=_PREC)
           + jnp.dot(neigh, Wfcb, precision=_PREC) + bfc)
    if apply_relu:
        rst = jnp.maximum(rst, 0.0)
    return jnp.dot(h_in, Wdst, precision=_PREC) + bdst + rst


def _tc_mid_body(apply_relu,
                 h_ref, aggP_ref, cd_ref, cs_ref,
                 Wfca_ref, Wfcb_ref, bfc_ref, Wdst_ref, bdst_ref,
                 g_ref, b_ref, Wsrc_ref, bsrc_ref,
                 ho_ref, y_ref, st_ref, hs_ref):
    p = pl.program_id(0)
    j = pl.program_id(1)

    @pl.when(p == 0)
    def _():
        h = _apply_math(h_ref[...], aggP_ref[...], cd_ref[...],
                        Wfca_ref[...], Wfcb_ref[...], bfc_ref[...],
                        Wdst_ref[...], bdst_ref[...], apply_relu)
        ho_ref[...] = h
        hs_ref[pl.ds(j * BR, BR), :] = h
        _stats_accum(st_ref, h, cs_ref[...], j == 0)
        y_ref[...] = jnp.zeros((BR, H), jnp.float32)

    @pl.when(p == 1)
    def _():
        h = hs_ref[pl.ds(j * BR, BR), :]
        ho_ref[...] = h
        y_ref[...] = _msg_math(h, st_ref[...], g_ref[...], b_ref[...],
                               Wsrc_ref[...], bsrc_ref[...])


def _tc_mid(apply_relu, h, aggP, cd, cs, Wfca, Wfcb, bfc, Wdst, bdst,
            g, b, Wsrc, bsrc):
    return pl.pallas_call(
        functools.partial(_tc_mid_body, apply_relu),
        grid=(2, NB),
        in_specs=[_BLK2(D),
                  pl.BlockSpec((2, BR, H), lambda p, j: (0, j, 0)),
                  _BLK2(1), _BLK2(1),
                  _FULL2((D, D)), _FULL2((H, D)), _FULL2((1, D)),
                  _FULL2((D, D)), _FULL2((1, D)),
                  _FULL2((D, 1)), _FULL2((D, 1)),
                  _FULL2((D, H)), _FULL2((1, H))],
        out_specs=[_BLK2(D), _BLK2(H)],
        out_shape=[jax.ShapeDtypeStruct((N, D), jnp.float32),
                   jax.ShapeDtypeStruct((N, H), jnp.float32)],
        scratch_shapes=[pltpu.VMEM((D, 8), jnp.float32),
                        pltpu.VMEM((N, D), jnp.float32)],
    )(h, aggP, cd, cs, Wfca, Wfcb, bfc, Wdst, bdst, g, b, Wsrc, bsrc)


def _tc_fin_body(h_ref, aggP_ref, cd_ref,
                 Wfca_ref, Wfcb_ref, bfc_ref, Wdst_ref, bdst_ref, out_ref):
    out_ref[...] = _apply_math(h_ref[...], aggP_ref[...], cd_ref[...],
                               Wfca_ref[...], Wfcb_ref[...], bfc_ref[...],
                               Wdst_ref[...], bdst_ref[...], False)


def _tc_fin(h, aggP, cd, Wfca, Wfcb, bfc, Wdst, bdst):
    blk = lambda w: pl.BlockSpec((BR, w), lambda j: (j, 0))
    full = lambda shape: pl.BlockSpec(shape, lambda j: tuple(0 for _ in shape))
    return pl.pallas_call(
        _tc_fin_body,
        grid=(NB,),
        in_specs=[blk(D),
                  pl.BlockSpec((2, BR, H), lambda j: (0, j, 0)),
                  blk(1),
                  full((D, 1)), full((H, 1)), full((1, 1)),
                  full((D, 1)), full((1, 1))],
        out_specs=blk(1),
        out_shape=jax.ShapeDtypeStruct((N, 1), jnp.float32),
    )(h, aggP, cd, Wfca, Wfcb, bfc, Wdst, bdst)


@jax.jit
def kernel(x, edge_index,
           bn_g1, bn_b1, Wsrc1, bsrc1, Wfc1, bfc1, Wdst1, bdst1,
           bn_g2, bn_b2, Wsrc2, bsrc2, Wfc2, bfc2, Wdst2, bdst2,
           bn_g3, bn_b3, Wsrc3, bsrc3, Wfc3, bfc3, Wdst3, bdst3):
    src = edge_index[0]
    dst = edge_index[1]
    pspread = jnp.arange(EPW_PAD - EPW, dtype=jnp.int32)[None, :] % 16
    spad = pspread
    tpad = TRASH + pspread
    spad = jnp.broadcast_to(spad, (NW, EPW_PAD - EPW))
    tpad = jnp.broadcast_to(tpad, (NW, EPW_PAD - EPW))
    srcs_g = jnp.concatenate([src.reshape(NW, EPW), spad], axis=1)
    srcs_g = srcs_g.reshape(NW, CH, CW)
    srcs_c = jnp.concatenate([src.reshape(NW, EPW), tpad], axis=1)
    srcs_c = srcs_c.reshape(NW, CH, CW)
    dsts_r = jnp.concatenate([dst.reshape(NW, EPW), tpad], axis=1)
    dsts_r = dsts_r.reshape(NW, CH, CW)

    zeros = jnp.zeros((NNP, H), jnp.float32)
    zeros_n = jnp.zeros((NNP, HC), jnp.float32)
    ones_rows = jnp.ones((CW, HC), jnp.float32)

    cntP = _sc_cnt(ones_rows, srcs_c, dsts_r, zeros_n)

    g1 = bn_g1.reshape(D, 1); b1 = bn_b1.reshape(D, 1)
    g2 = bn_g2.reshape(D, 1); b2 = bn_b2.reshape(D, 1)
    g3 = bn_g3.reshape(D, 1); b3 = bn_b3.reshape(D, 1)

    y1, cd, cs = _tc_pre(x, cntP, g1, b1, Wsrc1, bsrc1.reshape(1, H))
    aggP1 = _sc_agg(y1, srcs_g, dsts_r, zeros)
    h1, y2 = _tc_mid(True, x, aggP1, cd, cs,
                     Wfc1[0:D], Wfc1[D:], bfc1.reshape(1, D),
                     Wdst1, bdst1.reshape(1, D),
                     g2, b2, Wsrc2, bsrc2.reshape(1, H))
    aggP2 = _sc_agg(y2, srcs_g, dsts_r, zeros)
    h2, y3 = _tc_mid(False, h1, aggP2, cd, cs,
                     Wfc2[0:D], Wfc2[D:], bfc2.reshape(1, D),
                     Wdst2, bdst2.reshape(1, D),
                     g3, b3, Wsrc3, bsrc3.reshape(1, H))
    aggP3 = _sc_agg(y3, srcs_g, dsts_r, zeros)
    out = _tc_fin(h2, aggP3, cd,
                  Wfc3[0:D], Wfc3[D:], bfc3.reshape(1, 1),
                  Wdst3, bdst3.reshape(1, 1))
    return out


# speedup vs baseline: 1.2174x; 1.0457x over previous
import functools

import jax
import jax.numpy as jnp
from jax import lax
from jax.experimental import pallas as pl
from jax.experimental.pallas import tpu as pltpu
from jax.experimental.pallas import tpu_sc as plsc

N = 10000
E = 320000
D = 128
H = 32

NNP = 10016
TRASH = 10000
NW = 32
EPW = E // NW
CW = 128
CH = 80
EPW_PAD = CH * CW
RPS = NNP // 16

BR = 2000
NB = N // BR


YRS = N // 16


def _sc_agg_body(y_hbm, srcs_hbm, dsts_hbm, zeros_hbm, out_hbm,
                 src_v, dst_v, rows_v, y_sh, acc_sh):
    cid = lax.axis_index("c")
    sid = lax.axis_index("s")
    wid = sid * 2 + cid
    r0 = sid * RPS
    pltpu.sync_copy(zeros_hbm.at[pl.ds(r0, RPS)], acc_sh.at[pl.ds(r0, RPS)])
    pltpu.sync_copy(y_hbm.at[pl.ds(sid * YRS, YRS)],
                    y_sh.at[pl.ds(sid * YRS, YRS)])
    pltpu.sync_copy(srcs_hbm.at[wid], src_v)
    pltpu.sync_copy(dsts_hbm.at[wid], dst_v)
    plsc.subcore_barrier()

    def body(j, carry):
        pltpu.sync_copy(y_sh.at[src_v.at[j]], rows_v)
        pltpu.sync_copy(rows_v, acc_sh.at[dst_v.at[j]], add=True)
        return carry

    lax.fori_loop(0, CH, body, 0)
    plsc.subcore_barrier()
    pltpu.sync_copy(acc_sh.at[pl.ds(r0, RPS)],
                    out_hbm.at[cid, pl.ds(r0, RPS)])


_sc_agg = functools.partial(
    pl.kernel,
    mesh=plsc.VectorSubcoreMesh(core_axis_name="c", subcore_axis_name="s"),
    out_type=jax.ShapeDtypeStruct((2, NNP, H), jnp.float32),
    scratch_types=[
        pltpu.VMEM((CH, CW), jnp.int32),
        pltpu.VMEM((CH, CW), jnp.int32),
        pltpu.VMEM((CW, H), jnp.float32),
        pltpu.VMEM_SHARED((NNP, H), jnp.float32),
        pltpu.VMEM_SHARED((NNP, H), jnp.float32),
    ],
    compiler_params=pltpu.CompilerParams(use_tc_tiling_on_sc=False),
)(_sc_agg_body)


HC = 8


def _sc_cnt_body(ones_hbm, srcs_hbm, dsts_hbm, zeros_hbm, out_hbm,
                 src_v, dst_v, ones_v, accd_sh, accs_sh, sem):
    cid = lax.axis_index("c")
    sid = lax.axis_index("s")
    wid = sid * 2 + cid
    r0 = sid * RPS
    pltpu.sync_copy(zeros_hbm.at[pl.ds(r0, RPS)], accd_sh.at[pl.ds(r0, RPS)])
    pltpu.sync_copy(zeros_hbm.at[pl.ds(r0, RPS)], accs_sh.at[pl.ds(r0, RPS)])
    pltpu.sync_copy(srcs_hbm.at[wid], src_v)
    pltpu.sync_copy(dsts_hbm.at[wid], dst_v)
    pltpu.sync_copy(ones_hbm, ones_v)
    plsc.subcore_barrier()

    def body(j, carry):
        d1 = pltpu.async_copy(ones_v, accd_sh.at[dst_v.at[j]], sem, add=True)
        d2 = pltpu.async_copy(ones_v, accs_sh.at[src_v.at[j]], sem, add=True)
        d1.wait()
        d2.wait()
        return carry

    lax.fori_loop(0, CH, body, 0)
    plsc.subcore_barrier()
    pltpu.sync_copy(accd_sh.at[pl.ds(r0, RPS)],
                    out_hbm.at[cid, 0, pl.ds(r0, RPS)])
    pltpu.sync_copy(accs_sh.at[pl.ds(r0, RPS)],
                    out_hbm.at[cid, 1, pl.ds(r0, RPS)])


_sc_cnt = functools.partial(
    pl.kernel,
    mesh=plsc.VectorSubcoreMesh(core_axis_name="c", subcore_axis_name="s"),
    out_type=jax.ShapeDtypeStruct((2, 2, NNP, HC), jnp.float32),
    scratch_types=[
        pltpu.VMEM((CH, CW), jnp.int32),
        pltpu.VMEM((CH, CW), jnp.int32),
        pltpu.VMEM((CW, HC), jnp.float32),
        pltpu.VMEM_SHARED((NNP, HC), jnp.float32),
        pltpu.VMEM_SHARED((NNP, HC), jnp.float32),
        pltpu.SemaphoreType.DMA,
    ],
    compiler_params=pltpu.CompilerParams(use_tc_tiling_on_sc=False),
)(_sc_cnt_body)


_PREC = lax.Precision.DEFAULT
_FULL2 = lambda shape: pl.BlockSpec(shape, lambda p, j: tuple(0 for _ in shape))
_BLK2 = lambda w: pl.BlockSpec((BR, w), lambda p, j: (j, 0))


def _stats_accum(st_ref, h, cs, is_first):
    @pl.when(is_first)
    def _():
        st_ref[...] = jnp.zeros((D, 8), jnp.float32)
    s1 = lax.dot_general(h, cs, (((0,), (0,)), ((), ())), precision=_PREC)
    s2 = lax.dot_general(h * h, cs, (((0,), (0,)), ((), ())), precision=_PREC)
    st_ref[:, 0:1] += s1
    st_ref[:, 1:2] += s2


def _msg_math(h, st, g, b, Wsrc, bsrc):
    mu = st[:, 0:1] * (1.0 / E)
    var = st[:, 1:2] * (1.0 / E) - mu * mu
    scale = g * lax.rsqrt(var + 1e-5)
    shift = b - mu * scale
    Wp = scale * Wsrc
    bp = lax.dot_general(shift, Wsrc, (((0,), (0,)), ((), ())),
                         precision=_PREC) + bsrc
    z = jnp.dot(h, Wp, precision=_PREC) + bp
    return z * 0.5 * (1.0 + lax.erf(z * 0.7071067811865476))


def _tc_pre_body(x_ref, cntP_ref, g_ref, b_ref, Wsrc_ref, bsrc_ref,
                 y_ref, cd_ref, cs_ref, st_ref):
    p = pl.program_id(0)
    j = pl.program_id(1)
    cd = cntP_ref[0, 0, :, 0:1] + cntP_ref[1, 0, :, 0:1]
    cs = cntP_ref[0, 1, :, 0:1] + cntP_ref[1, 1, :, 0:1]
    cd_ref[...] = cd
    cs_ref[...] = cs

    @pl.when(p == 0)
    def _():
        _stats_accum(st_ref, x_ref[...], cs, j == 0)

    @pl.when(p == 1)
    def _():
        y_ref[...] = _msg_math(x_ref[...], st_ref[...], g_ref[...],
                               b_ref[...], Wsrc_ref[...], bsrc_ref[...])


_idx_p0 = lambda p, j: (j * (1 - p) + (NB - 1) * p, 0)
_idx_p1 = lambda p, j: (j * p, 0)


def _tc_pre(x, cntP, g, b, Wsrc, bsrc):
    return pl.pallas_call(
        _tc_pre_body,
        grid=(2, NB),
        in_specs=[_BLK2(D),
                  pl.BlockSpec((2, 2, BR, HC), lambda p, j: (0, 0, j, 0)),
                  _FULL2((D, 1)), _FULL2((D, 1)),
                  _FULL2((D, H)), _FULL2((1, H))],
        out_specs=[pl.BlockSpec((BR, H), _idx_p1), _BLK2(1), _BLK2(1)],
        out_shape=[jax.ShapeDtypeStruct((N, H), jnp.float32),
                   jax.ShapeDtypeStruct((N, 1), jnp.float32),
                   jax.ShapeDtypeStruct((N, 1), jnp.float32)],
        scratch_shapes=[pltpu.VMEM((D, 8), jnp.float32)],
    )(x, cntP, g, b, Wsrc, bsrc)


def _apply_math(h_in, aggP, cd, Wfca, Wfcb, bfc, Wdst, bdst, apply_relu):
    agg = aggP[0] + aggP[1]
    neigh = agg / jnp.maximum(cd, 1.0)
    rst = (jnp.dot(h_in, Wfca, precision=_PREC)
           + jnp.dot(neigh, Wfcb, precision=_PREC) + bfc)
    if apply_relu:
        rst = jnp.maximum(rst, 0.0)
    return jnp.dot(h_in, Wdst, precision=_PREC) + bdst + rst


def _tc_mid_body(apply_relu,
                 h_ref, aggP_ref, cd_ref, cs_ref,
                 Wfca_ref, Wfcb_ref, bfc_ref, Wdst_ref, bdst_ref,
                 g_ref, b_ref, Wsrc_ref, bsrc_ref,
                 ho_ref, y_ref, st_ref, hs_ref):
    p = pl.program_id(0)
    j = pl.program_id(1)

    @pl.when(p == 0)
    def _():
        h = _apply_math(h_ref[...], aggP_ref[...], cd_ref[...],
                        Wfca_ref[...], Wfcb_ref[...], bfc_ref[...],
                        Wdst_ref[...], bdst_ref[...], apply_relu)
        ho_ref[...] = h
        hs_ref[pl.ds(j * BR, BR), :] = h
        _stats_accum(st_ref, h, cs_ref[...], j == 0)

    @pl.when(p == 1)
    def _():
        h = hs_ref[pl.ds(j * BR, BR), :]
        y_ref[...] = _msg_math(h, st_ref[...], g_ref[...], b_ref[...],
                               Wsrc_ref[...], bsrc_ref[...])


def _tc_mid(apply_relu, h, aggP, cd, cs, Wfca, Wfcb, bfc, Wdst, bdst,
            g, b, Wsrc, bsrc):
    blk_p0 = lambda w: pl.BlockSpec((BR, w), _idx_p0)
    return pl.pallas_call(
        functools.partial(_tc_mid_body, apply_relu),
        grid=(2, NB),
        in_specs=[blk_p0(D),
                  pl.BlockSpec((2, BR, H),
                               lambda p, j: (0,) + _idx_p0(p, j)),
                  blk_p0(1), blk_p0(1),
                  _FULL2((D, D)), _FULL2((H, D)), _FULL2((1, D)),
                  _FULL2((D, D)), _FULL2((1, D)),
                  _FULL2((D, 1)), _FULL2((D, 1)),
                  _FULL2((D, H)), _FULL2((1, H))],
        out_specs=[pl.BlockSpec((BR, D), _idx_p0),
                   pl.BlockSpec((BR, H), _idx_p1)],
        out_shape=[jax.ShapeDtypeStruct((N, D), jnp.float32),
                   jax.ShapeDtypeStruct((N, H), jnp.float32)],
        scratch_shapes=[pltpu.VMEM((D, 8), jnp.float32),
                        pltpu.VMEM((N, D), jnp.float32)],
    )(h, aggP, cd, cs, Wfca, Wfcb, bfc, Wdst, bdst, g, b, Wsrc, bsrc)


def _tc_fin_body(h_ref, aggP_ref, cd_ref,
                 Wfca_ref, Wfcb_ref, bfc_ref, Wdst_ref, bdst_ref, out_ref):
    out_ref[...] = _apply_math(h_ref[...], aggP_ref[...], cd_ref[...],
                               Wfca_ref[...], Wfcb_ref[...], bfc_ref[...],
                               Wdst_ref[...], bdst_ref[...], False)


def _tc_fin(h, aggP, cd, Wfca, Wfcb, bfc, Wdst, bdst):
    blk = lambda w: pl.BlockSpec((BR, w), lambda j: (j, 0))
    full = lambda shape: pl.BlockSpec(shape, lambda j: tuple(0 for _ in shape))
    return pl.pallas_call(
        _tc_fin_body,
        grid=(NB,),
        in_specs=[blk(D),
                  pl.BlockSpec((2, BR, H), lambda j: (0, j, 0)),
                  blk(1),
                  full((D, 1)), full((H, 1)), full((1, 1)),
                  full((D, 1)), full((1, 1))],
        out_specs=blk(1),
        out_shape=jax.ShapeDtypeStruct((N, 1), jnp.float32),
    )(h, aggP, cd, Wfca, Wfcb, bfc, Wdst, bdst)


@jax.jit
def kernel(x, edge_index,
           bn_g1, bn_b1, Wsrc1, bsrc1, Wfc1, bfc1, Wdst1, bdst1,
           bn_g2, bn_b2, Wsrc2, bsrc2, Wfc2, bfc2, Wdst2, bdst2,
           bn_g3, bn_b3, Wsrc3, bsrc3, Wfc3, bfc3, Wdst3, bdst3):
    src = edge_index[0]
    dst = edge_index[1]
    pspread = jnp.arange(EPW_PAD - EPW, dtype=jnp.int32)[None, :] % 16
    spad = pspread
    tpad = TRASH + pspread
    spad = jnp.broadcast_to(spad, (NW, EPW_PAD - EPW))
    tpad = jnp.broadcast_to(tpad, (NW, EPW_PAD - EPW))
    srcs_g = jnp.concatenate([src.reshape(NW, EPW), spad], axis=1)
    srcs_g = srcs_g.reshape(NW, CH, CW)
    srcs_c = jnp.concatenate([src.reshape(NW, EPW), tpad], axis=1)
    srcs_c = srcs_c.reshape(NW, CH, CW)
    dsts_r = jnp.concatenate([dst.reshape(NW, EPW), tpad], axis=1)
    dsts_r = dsts_r.reshape(NW, CH, CW)

    zeros = jnp.zeros((NNP, H), jnp.float32)
    zeros_n = jnp.zeros((NNP, HC), jnp.float32)
    ones_rows = jnp.ones((CW, HC), jnp.float32)

    cntP = _sc_cnt(ones_rows, srcs_c, dsts_r, zeros_n)

    g1 = bn_g1.reshape(D, 1); b1 = bn_b1.reshape(D, 1)
    g2 = bn_g2.reshape(D, 1); b2 = bn_b2.reshape(D, 1)
    g3 = bn_g3.reshape(D, 1); b3 = bn_b3.reshape(D, 1)

    y1, cd, cs = _tc_pre(x, cntP, g1, b1, Wsrc1, bsrc1.reshape(1, H))
    aggP1 = _sc_agg(y1, srcs_g, dsts_r, zeros)
    h1, y2 = _tc_mid(True, x, aggP1, cd, cs,
                     Wfc1[0:D], Wfc1[D:], bfc1.reshape(1, D),
                     Wdst1, bdst1.reshape(1, D),
                     g2, b2, Wsrc2, bsrc2.reshape(1, H))
    aggP2 = _sc_agg(y2, srcs_g, dsts_r, zeros)
    h2, y3 = _tc_mid(False, h1, aggP2, cd, cs,
                     Wfc2[0:D], Wfc2[D:], bfc2.reshape(1, D),
                     Wdst2, bdst2.reshape(1, D),
                     g3, b3, Wsrc3, bsrc3.reshape(1, H))
    aggP3 = _sc_agg(y3, srcs_g, dsts_r, zeros)
    out = _tc_fin(h2, aggP3, cd,
                  Wfc3[0:D], Wfc3[D:], bfc3.reshape(1, 1),
                  Wdst3, bdst3.reshape(1, 1))
    return out


# speedup vs baseline: 1.2191x; 1.0013x over previous
import functools

import jax
import jax.numpy as jnp
from jax import lax
from jax.experimental import pallas as pl
from jax.experimental.pallas import tpu as pltpu
from jax.experimental.pallas import tpu_sc as plsc

N = 10000
E = 320000
D = 128
H = 32

NP = 10000
NNP = 10016
TRASH = 10000
NW = 32
EPW = E // NW
CW = 128
CH = 80
EPW_PAD = CH * CW
RPS = NNP // 16

BR = 2000
NB = NP // BR


YRS = NP // 16


def _sc_agg_body(y_hbm, srcs_hbm, dsts_hbm, zeros_hbm, out_hbm,
                 src_v, dst_v, rows_v, y_sh, acc_sh):
    cid = lax.axis_index("c")
    sid = lax.axis_index("s")
    wid = sid * 2 + cid
    r0 = sid * RPS
    pltpu.sync_copy(zeros_hbm.at[pl.ds(r0, RPS)], acc_sh.at[pl.ds(r0, RPS)])
    pltpu.sync_copy(y_hbm.at[pl.ds(sid * YRS, YRS)],
                    y_sh.at[pl.ds(sid * YRS, YRS)])
    pltpu.sync_copy(srcs_hbm.at[wid], src_v)
    pltpu.sync_copy(dsts_hbm.at[wid], dst_v)
    plsc.subcore_barrier()

    def body(j, carry):
        pltpu.sync_copy(y_sh.at[src_v.at[j]], rows_v)
        pltpu.sync_copy(rows_v, acc_sh.at[dst_v.at[j]], add=True)
        return carry

    lax.fori_loop(0, CH, body, 0)
    plsc.subcore_barrier()
    pltpu.sync_copy(acc_sh.at[pl.ds(r0, RPS)],
                    out_hbm.at[cid, pl.ds(r0, RPS)])


_sc_agg = functools.partial(
    pl.kernel,
    mesh=plsc.VectorSubcoreMesh(core_axis_name="c", subcore_axis_name="s"),
    out_type=jax.ShapeDtypeStruct((2, NNP, H), jnp.float32),
    scratch_types=[
        pltpu.VMEM((CH, CW), jnp.int32),
        pltpu.VMEM((CH, CW), jnp.int32),
        pltpu.VMEM((CW, H), jnp.float32),
        pltpu.VMEM_SHARED((NNP, H), jnp.float32),
        pltpu.VMEM_SHARED((NNP, H), jnp.float32),
    ],
    compiler_params=pltpu.CompilerParams(use_tc_tiling_on_sc=False),
)(_sc_agg_body)


HC = 8


def _sc_cnt_body(ones_hbm, srcs_hbm, dsts_hbm, zeros_hbm, out_hbm,
                 src_v, dst_v, ones_v, accd_sh, accs_sh, sem):
    cid = lax.axis_index("c")
    sid = lax.axis_index("s")
    wid = sid * 2 + cid
    r0 = sid * RPS
    pltpu.sync_copy(zeros_hbm.at[pl.ds(r0, RPS)], accd_sh.at[pl.ds(r0, RPS)])
    pltpu.sync_copy(zeros_hbm.at[pl.ds(r0, RPS)], accs_sh.at[pl.ds(r0, RPS)])
    pltpu.sync_copy(srcs_hbm.at[wid], src_v)
    pltpu.sync_copy(dsts_hbm.at[wid], dst_v)
    pltpu.sync_copy(ones_hbm, ones_v)
    plsc.subcore_barrier()

    def body(j, carry):
        d1 = pltpu.async_copy(ones_v, accd_sh.at[dst_v.at[j]], sem, add=True)
        d2 = pltpu.async_copy(ones_v, accs_sh.at[src_v.at[j]], sem, add=True)
        d1.wait()
        d2.wait()
        return carry

    lax.fori_loop(0, CH, body, 0)
    plsc.subcore_barrier()
    pltpu.sync_copy(accd_sh.at[pl.ds(r0, RPS)],
                    out_hbm.at[cid, 0, pl.ds(r0, RPS)])
    pltpu.sync_copy(accs_sh.at[pl.ds(r0, RPS)],
                    out_hbm.at[cid, 1, pl.ds(r0, RPS)])


_sc_cnt = functools.partial(
    pl.kernel,
    mesh=plsc.VectorSubcoreMesh(core_axis_name="c", subcore_axis_name="s"),
    out_type=jax.ShapeDtypeStruct((2, 2, NNP, HC), jnp.float32),
    scratch_types=[
        pltpu.VMEM((CH, CW), jnp.int32),
        pltpu.VMEM((CH, CW), jnp.int32),
        pltpu.VMEM((CW, HC), jnp.float32),
        pltpu.VMEM_SHARED((NNP, HC), jnp.float32),
        pltpu.VMEM_SHARED((NNP, HC), jnp.float32),
        pltpu.SemaphoreType.DMA,
    ],
    compiler_params=pltpu.CompilerParams(use_tc_tiling_on_sc=False),
)(_sc_cnt_body)


_PREC = lax.Precision.DEFAULT
_FULL2 = lambda shape: pl.BlockSpec(shape, lambda p, j: tuple(0 for _ in shape))
_BLK2 = lambda w: pl.BlockSpec((BR, w), lambda p, j: (j, 0))


def _stats_accum(st_ref, h, cs, is_first):
    @pl.when(is_first)
    def _():
        st_ref[...] = jnp.zeros((D, 8), jnp.float32)
    s1 = lax.dot_general(h, cs, (((0,), (0,)), ((), ())), precision=_PREC)
    s2 = lax.dot_general(h * h, cs, (((0,), (0,)), ((), ())), precision=_PREC)
    st_ref[:, 0:1] += s1
    st_ref[:, 1:2] += s2


def _msg_math(h, st, g, b, Wsrc, bsrc):
    mu = st[:, 0:1] * (1.0 / E)
    var = st[:, 1:2] * (1.0 / E) - mu * mu
    scale = g * lax.rsqrt(var + 1e-5)
    shift = b - mu * scale
    Wp = scale * Wsrc
    bp = lax.dot_general(shift, Wsrc, (((0,), (0,)), ((), ())),
                         precision=_PREC) + bsrc
    z = jnp.dot(h, Wp, precision=_PREC) + bp
    return z * 0.5 * (1.0 + lax.erf(z * 0.7071067811865476))


def _tc_pre_body(x_ref, cntP_ref, g_ref, b_ref, Wsrc_ref, bsrc_ref,
                 y_ref, cd_ref, cs_ref, st_ref):
    p = pl.program_id(0)
    j = pl.program_id(1)
    row = j * BR + lax.broadcasted_iota(jnp.int32, (BR, 1), 0)
    real = (row < N).astype(jnp.float32)
    cd = (cntP_ref[0, 0] + cntP_ref[1, 0])[:, 0:1] * real
    cs = (cntP_ref[0, 1] + cntP_ref[1, 1])[:, 0:1] * real
    cd_ref[...] = cd
    cs_ref[...] = cs

    @pl.when(p == 0)
    def _():
        _stats_accum(st_ref, x_ref[...], cs, j == 0)

    @pl.when(p == 1)
    def _():
        y = _msg_math(x_ref[...], st_ref[...], g_ref[...],
                      b_ref[...], Wsrc_ref[...], bsrc_ref[...])
        y_ref[...] = y


_idx_p0 = lambda p, j: (j * (1 - p) + (NB - 1) * p, 0)
_idx_p1 = lambda p, j: (j * p, 0)


def _tc_pre(x, cntP, g, b, Wsrc, bsrc):
    return pl.pallas_call(
        _tc_pre_body,
        grid=(2, NB),
        in_specs=[_BLK2(D),
                  pl.BlockSpec((2, 2, BR, HC), lambda p, j: (0, 0, j, 0)),
                  _FULL2((D, 1)), _FULL2((D, 1)),
                  _FULL2((D, H)), _FULL2((1, H))],
        out_specs=[pl.BlockSpec((BR, H), _idx_p1), _BLK2(1), _BLK2(1)],
        out_shape=[jax.ShapeDtypeStruct((NP, H), jnp.float32),
                   jax.ShapeDtypeStruct((NP, 1), jnp.float32),
                   jax.ShapeDtypeStruct((NP, 1), jnp.float32)],
        scratch_shapes=[pltpu.VMEM((D, 8), jnp.float32)],
    )(x, cntP, g, b, Wsrc, bsrc)


def _apply_math(h_in, agg, cd, Wfca, Wfcb, bfc, Wdst, bdst, apply_relu):
    neigh = agg / jnp.maximum(cd, 1.0)
    rst = (jnp.dot(h_in, Wfca, precision=_PREC)
           + jnp.dot(neigh, Wfcb, precision=_PREC) + bfc)
    if apply_relu:
        rst = jnp.maximum(rst, 0.0)
    return jnp.dot(h_in, Wdst, precision=_PREC) + bdst + rst


def _tc_mid_body(apply_relu,
                 h_ref, aggP_ref, cd_ref, cs_ref,
                 Wfca_ref, Wfcb_ref, bfc_ref, Wdst_ref, bdst_ref,
                 g_ref, b_ref, Wsrc_ref, bsrc_ref,
                 ho_ref, y_ref, st_ref, hs_ref):
    p = pl.program_id(0)
    j = pl.program_id(1)

    @pl.when(p == 0)
    def _():
        agg = aggP_ref[0] + aggP_ref[1]
        h = _apply_math(h_ref[...], agg, cd_ref[...],
                        Wfca_ref[...], Wfcb_ref[...], bfc_ref[...],
                        Wdst_ref[...], bdst_ref[...], apply_relu)
        ho_ref[...] = h
        hs_ref[pl.ds(j * BR, BR), :] = h
        _stats_accum(st_ref, h, cs_ref[...], j == 0)

    @pl.when(p == 1)
    def _():
        h = hs_ref[pl.ds(j * BR, BR), :]
        y = _msg_math(h, st_ref[...], g_ref[...], b_ref[...],
                      Wsrc_ref[...], bsrc_ref[...])
        y_ref[...] = y


def _tc_mid(apply_relu, h, aggP, cd, cs, Wfca, Wfcb, bfc, Wdst, bdst,
            g, b, Wsrc, bsrc):
    blk_p0 = lambda w: pl.BlockSpec((BR, w), _idx_p0)
    return pl.pallas_call(
        functools.partial(_tc_mid_body, apply_relu),
        grid=(2, NB),
        in_specs=[blk_p0(D),
                  pl.BlockSpec((2, BR, H),
                               lambda p, j: (0,) + _idx_p0(p, j)),
                  blk_p0(1), blk_p0(1),
                  _FULL2((D, D)), _FULL2((H, D)), _FULL2((1, D)),
                  _FULL2((D, D)), _FULL2((1, D)),
                  _FULL2((D, 1)), _FULL2((D, 1)),
                  _FULL2((D, H)), _FULL2((1, H))],
        out_specs=[pl.BlockSpec((BR, D), _idx_p0),
                   pl.BlockSpec((BR, H), _idx_p1)],
        out_shape=[jax.ShapeDtypeStruct((NP, D), jnp.float32),
                   jax.ShapeDtypeStruct((NP, H), jnp.float32)],
        scratch_shapes=[pltpu.VMEM((D, 8), jnp.float32),
                        pltpu.VMEM((NP, D), jnp.float32)],
    )(h, aggP, cd, cs, Wfca, Wfcb, bfc, Wdst, bdst, g, b, Wsrc, bsrc)


def _tc_fin_body(h_ref, aggP_ref, cd_ref,
                 Wfca_ref, Wfcb_ref, bfc_ref, Wdst_ref, bdst_ref, out_ref):
    agg = aggP_ref[0] + aggP_ref[1]
    out_ref[...] = _apply_math(h_ref[...], agg, cd_ref[...],
                               Wfca_ref[...], Wfcb_ref[...], bfc_ref[...],
                               Wdst_ref[...], bdst_ref[...], False)


def _tc_fin(h, aggP, cd, Wfca, Wfcb, bfc, Wdst, bdst):
    blk = lambda w: pl.BlockSpec((BR, w), lambda j: (j, 0))
    full = lambda shape: pl.BlockSpec(shape, lambda j: tuple(0 for _ in shape))
    return pl.pallas_call(
        _tc_fin_body,
        grid=(NB,),
        in_specs=[blk(D),
                  pl.BlockSpec((2, BR, H), lambda j: (0, j, 0)),
                  blk(1),
                  full((D, 1)), full((H, 1)), full((1, 1)),
                  full((D, 1)), full((1, 1))],
        out_specs=blk(1),
        out_shape=jax.ShapeDtypeStruct((N, 1), jnp.float32),
    )(h, aggP, cd, Wfca, Wfcb, bfc, Wdst, bdst)


@jax.jit
def kernel(x, edge_index,
           bn_g1, bn_b1, Wsrc1, bsrc1, Wfc1, bfc1, Wdst1, bdst1,
           bn_g2, bn_b2, Wsrc2, bsrc2, Wfc2, bfc2, Wdst2, bdst2,
           bn_g3, bn_b3, Wsrc3, bsrc3, Wfc3, bfc3, Wdst3, bdst3):
    src = edge_index[0]
    dst = edge_index[1]
    pspread = jnp.arange(EPW_PAD - EPW, dtype=jnp.int32)[None, :] % 16
    spad = pspread
    tpad = TRASH + pspread
    spad = jnp.broadcast_to(spad, (NW, EPW_PAD - EPW))
    tpad = jnp.broadcast_to(tpad, (NW, EPW_PAD - EPW))
    srcs_g = jnp.concatenate([src.reshape(NW, EPW), spad], axis=1)
    srcs_g = srcs_g.reshape(NW, CH, CW)
    srcs_c = jnp.concatenate([src.reshape(NW, EPW), tpad], axis=1)
    srcs_c = srcs_c.reshape(NW, CH, CW)
    dsts_r = jnp.concatenate([dst.reshape(NW, EPW), tpad], axis=1)
    dsts_r = dsts_r.reshape(NW, CH, CW)

    zeros = jnp.zeros((NNP, H), jnp.float32)
    zeros_n = jnp.zeros((NNP, HC), jnp.float32)
    ones_rows = jnp.ones((CW, HC), jnp.float32)

    cntP = _sc_cnt(ones_rows, srcs_c, dsts_r, zeros_n)

    g1 = bn_g1.reshape(D, 1); b1 = bn_b1.reshape(D, 1)
    g2 = bn_g2.reshape(D, 1); b2 = bn_b2.reshape(D, 1)
    g3 = bn_g3.reshape(D, 1); b3 = bn_b3.reshape(D, 1)

    y1, cd, cs = _tc_pre(x, cntP, g1, b1, Wsrc1, bsrc1.reshape(1, H))
    aggP1 = _sc_agg(y1, srcs_g, dsts_r, zeros)
    h1, y2 = _tc_mid(True, x, aggP1, cd, cs,
                     Wfc1[0:D], Wfc1[D:], bfc1.reshape(1, D),
                     Wdst1, bdst1.reshape(1, D),
                     g2, b2, Wsrc2, bsrc2.reshape(1, H))
    aggP2 = _sc_agg(y2, srcs_g, dsts_r, zeros)
    h2, y3 = _tc_mid(False, h1, aggP2, cd, cs,
                     Wfc2[0:D], Wfc2[D:], bfc2.reshape(1, D),
                     Wdst2, bdst2.reshape(1, D),
                     g3, b3, Wsrc3, bsrc3.reshape(1, H))
    aggP3 = _sc_agg(y3, srcs_g, dsts_r, zeros)
    out = _tc_fin(h2, aggP3, cd,
                  Wfc3[0:D], Wfc3[D:], bfc3.reshape(1, 1),
                  Wdst3, bdst3.reshape(1, 1))
    return out
